# Initial kernel scaffold; baseline (speedup 1.0000x reference)
import jax
import jax.numpy as jnp
from jax.experimental import pallas as pl


def kernel(pos, edge_shift, lattice, edge_index, batch_idx, edge_idx_ji, edge_idx_kj):
    raise NotImplementedError("write your pallas kernel here")


# speedup vs baseline: 12.8667x; 12.8667x over previous
import functools

import jax
import jax.numpy as jnp
from jax import lax
from jax.experimental import pallas as pl
from jax.experimental.pallas import tpu as pltpu
from jax.experimental.pallas import tpu_sc as plsc

_NC = 2
_NS = 16
_NW = _NC * _NS
_L = 16

_HALF_PI = 1.5707963267948966
_PI = 3.141592653589793
_TINY = 1e-30

_ATAN_C = (0.99997726, -0.33262347, 0.19354346, -0.11643287, 0.05265332,
           -0.01172120)


def _rsqrt(x):
    i = plsc.bitcast(x, jnp.int32)
    i = jnp.int32(0x5F3759DF) - lax.shift_right_arithmetic(i, 1)
    y = plsc.bitcast(i, jnp.float32)
    for _ in range(2):
        y = y * (1.5 - 0.5 * x * y * y)
    return y


def _sqrt(x):
    xc = jnp.maximum(x, _TINY)
    return x * _rsqrt(xc)


def _atan2_pos(y, x):
    ax = jnp.abs(x)
    mn = jnp.minimum(y, ax)
    mx = jnp.maximum(y, ax)
    q = mn / jnp.maximum(mx, _TINY)
    q2 = q * q
    p = jnp.float32(_ATAN_C[5])
    for k in (4, 3, 2, 1, 0):
        p = p * q2 + _ATAN_C[k]
    p = p * q
    r = jnp.where(y > ax, _HALF_PI - p, p)
    return jnp.where(x < 0.0, _PI - r, r)


def _make_edge_kernel(E, EC):
    EPW = E // _NW
    n_chunks = EPW // EC
    assert EPW % EC == 0 and EC % _L == 0

    mesh = plsc.VectorSubcoreMesh(core_axis_name="c", subcore_axis_name="s")
    fvec = jax.ShapeDtypeStruct((E,), jnp.float32)

    @functools.partial(
        pl.kernel,
        mesh=mesh,
        compiler_params=pltpu.CompilerParams(needs_layout_passes=False),
        out_type=(fvec, fvec, fvec, fvec),
        scratch_types=[
            pltpu.VMEM((EC,), jnp.int32),
            pltpu.VMEM((EC,), jnp.int32),
            pltpu.VMEM((EC,), jnp.int32),
            pltpu.VMEM((3 * EC,), jnp.float32),
            pltpu.VMEM((EC,), jnp.float32),
            pltpu.VMEM((EC,), jnp.float32),
            pltpu.VMEM((EC,), jnp.float32),
            pltpu.VMEM((EC,), jnp.float32),
            pltpu.VMEM((EC,), jnp.float32),
            pltpu.VMEM((EC,), jnp.float32),
            pltpu.VMEM((EC,), jnp.float32),
            pltpu.VMEM((EC,), jnp.float32),
            pltpu.VMEM((EC,), jnp.float32),
            pltpu.VMEM((EC,), jnp.float32),
            pltpu.VMEM((144,), jnp.float32),
            pltpu.SemaphoreType.DMA,
        ],
    )
    def edge_k(dst_hbm, src_hbm, shift_hbm, px_hbm, py_hbm, pz_hbm,
               batch_hbm, lat_hbm,
               vx_hbm, vy_hbm, vz_hbm, dist_hbm,
               dst_v, src_v, batch_v, shift_v,
               pdx_v, pdy_v, pdz_v, psx_v, psy_v, psz_v,
               vx_v, vy_v, vz_v, dist_v, lat_v, sem):
        wid = lax.axis_index("s") * _NC + lax.axis_index("c")
        pltpu.sync_copy(lat_hbm, lat_v)
        lanes = lax.iota(jnp.int32, 16)

        def chunk_body(i, carry):
            base = wid * EPW + i * EC
            pltpu.sync_copy(dst_hbm.at[pl.ds(base, EC)], dst_v)
            pltpu.sync_copy(src_hbm.at[pl.ds(base, EC)], src_v)
            pltpu.sync_copy(shift_hbm.at[pl.ds(3 * base, 3 * EC)], shift_v)
            cps = [
                pltpu.async_copy(px_hbm.at[dst_v], pdx_v, sem),
                pltpu.async_copy(py_hbm.at[dst_v], pdy_v, sem),
                pltpu.async_copy(pz_hbm.at[dst_v], pdz_v, sem),
                pltpu.async_copy(px_hbm.at[src_v], psx_v, sem),
                pltpu.async_copy(py_hbm.at[src_v], psy_v, sem),
                pltpu.async_copy(pz_hbm.at[src_v], psz_v, sem),
                pltpu.async_copy(batch_hbm.at[src_v], batch_v, sem),
            ]
            for cp in cps:
                cp.wait()

            def vec_body(v, carry2):
                s = v * _L
                row3 = (s + lanes) * 3
                shx = plsc.load_gather(shift_v, [row3])
                shy = plsc.load_gather(shift_v, [row3 + 1])
                shz = plsc.load_gather(shift_v, [row3 + 2])
                b9 = batch_v[pl.ds(s, _L)] * 9
                l00 = plsc.load_gather(lat_v, [b9])
                l01 = plsc.load_gather(lat_v, [b9 + 1])
                l02 = plsc.load_gather(lat_v, [b9 + 2])
                l10 = plsc.load_gather(lat_v, [b9 + 3])
                l11 = plsc.load_gather(lat_v, [b9 + 4])
                l12 = plsc.load_gather(lat_v, [b9 + 5])
                l20 = plsc.load_gather(lat_v, [b9 + 6])
                l21 = plsc.load_gather(lat_v, [b9 + 7])
                l22 = plsc.load_gather(lat_v, [b9 + 8])
                scx = shx * l00 + shy * l10 + shz * l20
                scy = shx * l01 + shy * l11 + shz * l21
                scz = shx * l02 + shy * l12 + shz * l22
                vx = pdx_v[pl.ds(s, _L)] - psx_v[pl.ds(s, _L)] + scx
                vy = pdy_v[pl.ds(s, _L)] - psy_v[pl.ds(s, _L)] + scy
                vz = pdz_v[pl.ds(s, _L)] - psz_v[pl.ds(s, _L)] + scz
                vv = vx * vx + vy * vy + vz * vz
                vx_v[pl.ds(s, _L)] = vx
                vy_v[pl.ds(s, _L)] = vy
                vz_v[pl.ds(s, _L)] = vz
                dist_v[pl.ds(s, _L)] = _sqrt(vv)
                return carry2

            lax.fori_loop(0, EC // _L, vec_body, 0)
            pltpu.sync_copy(vx_v, vx_hbm.at[pl.ds(base, EC)])
            pltpu.sync_copy(vy_v, vy_hbm.at[pl.ds(base, EC)])
            pltpu.sync_copy(vz_v, vz_hbm.at[pl.ds(base, EC)])
            pltpu.sync_copy(dist_v, dist_hbm.at[pl.ds(base, EC)])
            return carry

        lax.fori_loop(0, n_chunks, chunk_body, 0)

    return edge_k


def _make_angle_kernel(T, TC):
    TPW = T // _NW
    n_chunks = TPW // TC
    assert TPW % TC == 0 and TC % _L == 0

    mesh = plsc.VectorSubcoreMesh(core_axis_name="c", subcore_axis_name="s")

    @functools.partial(
        pl.kernel,
        mesh=mesh,
        compiler_params=pltpu.CompilerParams(needs_layout_passes=False),
        out_type=jax.ShapeDtypeStruct((T,), jnp.float32),
        scratch_types=[
            pltpu.VMEM((TC,), jnp.int32),
            pltpu.VMEM((TC,), jnp.int32),
            pltpu.VMEM((TC,), jnp.float32),
            pltpu.VMEM((TC,), jnp.float32),
            pltpu.VMEM((TC,), jnp.float32),
            pltpu.VMEM((TC,), jnp.float32),
            pltpu.VMEM((TC,), jnp.float32),
            pltpu.VMEM((TC,), jnp.float32),
            pltpu.VMEM((TC,), jnp.float32),
            pltpu.SemaphoreType.DMA,
        ],
    )
    def angle_k(vx_hbm, vy_hbm, vz_hbm, ji_hbm, kj_hbm, ang_hbm,
                ji_v, kj_v, ax_v, ay_v, az_v, bx_v, by_v, bz_v, ang_v, sem):
        wid = lax.axis_index("s") * _NC + lax.axis_index("c")

        def chunk_body(i, carry):
            base = wid * TPW + i * TC
            pltpu.sync_copy(ji_hbm.at[pl.ds(base, TC)], ji_v)
            pltpu.sync_copy(kj_hbm.at[pl.ds(base, TC)], kj_v)
            cps = [
                pltpu.async_copy(vx_hbm.at[ji_v], ax_v, sem),
                pltpu.async_copy(vy_hbm.at[ji_v], ay_v, sem),
                pltpu.async_copy(vz_hbm.at[ji_v], az_v, sem),
                pltpu.async_copy(vx_hbm.at[kj_v], bx_v, sem),
                pltpu.async_copy(vy_hbm.at[kj_v], by_v, sem),
                pltpu.async_copy(vz_hbm.at[kj_v], bz_v, sem),
            ]
            for cp in cps:
                cp.wait()

            def vec_body(v, carry2):
                s = v * _L
                ax = ax_v[pl.ds(s, _L)]
                ay = ay_v[pl.ds(s, _L)]
                az = az_v[pl.ds(s, _L)]
                bx = bx_v[pl.ds(s, _L)]
                by = by_v[pl.ds(s, _L)]
                bz = bz_v[pl.ds(s, _L)]
                inner = ax * bx + ay * by + az * bz
                cx = ay * bz - az * by
                cy = az * bx - ax * bz
                cz = ax * by - ay * bx
                cc = cx * cx + cy * cy + cz * cz
                outter = _sqrt(cc)
                ang_v[pl.ds(s, _L)] = _atan2_pos(outter, inner)
                return carry2

            lax.fori_loop(0, TC // _L, vec_body, 0)
            pltpu.sync_copy(ang_v, ang_hbm.at[pl.ds(base, TC)])
            return carry

        lax.fori_loop(0, n_chunks, chunk_body, 0)

    return angle_k


def kernel(pos, edge_shift, lattice, edge_index, batch_idx, edge_idx_ji,
           edge_idx_kj):
    E = edge_shift.shape[0]
    T = edge_idx_ji.shape[0]
    edge_dst = edge_index[0]
    edge_src = edge_index[1]
    pos_t = pos.T
    lat_flat = lattice.reshape(-1)
    shift_flat = edge_shift.reshape(-1)

    edge_k = _make_edge_kernel(E, EC=4000)
    angle_k = _make_angle_kernel(T, TC=8000)

    vx, vy, vz, edge_dist = edge_k(edge_dst, edge_src, shift_flat,
                                   pos_t[0], pos_t[1], pos_t[2],
                                   batch_idx, lat_flat)
    angles = angle_k(vx, vy, vz, edge_idx_ji, edge_idx_kj)
    return edge_dist, angles
